# baseline (speedup 1.0000x reference)
import jax
import jax.numpy as jnp
from jax import lax
from jax.experimental import pallas as pl
from jax.experimental.pallas import tpu as pltpu
from jax.experimental.pallas import tpu_sc as plsc

N_FIELDS = 26
VOCAB = 100000
EMB = 16
NUM_DENSE = 13
BATCH = 16384

ROWS_PER_LINE = 8
LINES_PER_FIELD = VOCAB // ROWS_PER_LINE
FIELD_STRIDE = 12504
LINE_W = ROWS_PER_LINE * EMB

NW = 32
SPW = BATCH // NW
CH = 128
NCH = SPW // CH

VCH = 512
NFULL = VOCAB // VCH
VTAIL = VOCAB - NFULL * VCH
BLOCKS_PER_FIELD = NFULL + 1
TOTAL_BLOCKS = N_FIELDS * BLOCKS_PER_FIELD


def _transpose_body(tabT_hbm, lines_hbm, in_v, out_v, in_t, out_t, sem):
    cid = lax.axis_index("c")
    sid = lax.axis_index("s")
    wid = sid * 2 + cid
    lane = lax.iota(jnp.int32, 16)
    nblk = jnp.where(wid < TOTAL_BLOCKS % NW,
                     TOTAL_BLOCKS // NW + 1, TOTAL_BLOCKS // NW)

    def blk_body(i, _):
        blk = wid + i * NW
        f = blk // BLOCKS_PER_FIELD
        b = blk % BLOCKS_PER_FIELD
        v0 = b * VCH
        line0 = f * FIELD_STRIDE + b * (VCH // ROWS_PER_LINE)

        @pl.when(b < NFULL)
        def _full():
            pltpu.sync_copy(tabT_hbm.at[f, :, pl.ds(v0, VCH)], in_v)

            def vloop(v8, _):
                for k in range(ROWS_PER_LINE):
                    vals = plsc.load_gather(
                        in_v, [lane, jnp.full((16,), v8 * 8 + k, jnp.int32)])
                    out_v[v8, pl.ds(k * EMB, EMB)] = vals
                return 0

            lax.fori_loop(0, VCH // ROWS_PER_LINE, vloop, 0)
            pltpu.sync_copy(out_v,
                            lines_hbm.at[pl.ds(line0, VCH // ROWS_PER_LINE), :])

        @pl.when(b == NFULL)
        def _tail():
            pltpu.sync_copy(tabT_hbm.at[f, :, pl.ds(v0, 2 * LINE_W)], in_t)

            def vloop(v8, _):
                for k in range(ROWS_PER_LINE):
                    vals = plsc.load_gather(
                        in_t, [lane, jnp.full((16,), v8 * 8 + k, jnp.int32)])
                    out_t[v8, pl.ds(k * EMB, EMB)] = vals
                return 0

            lax.fori_loop(0, VTAIL // ROWS_PER_LINE, vloop, 0)
            pltpu.sync_copy(out_t, lines_hbm.at[pl.ds(line0, 24), :])

        return 0

    lax.fori_loop(0, nblk, blk_body, 0)


def _sc_body(tab_hbm, xs_hbm, xd_hbm, w_hbm, out_hbm,
             xs_v, xd_v, idx_v, off_v, buf0, buf1, s_v, ss_v, w_v, t_v, out_v,
             sem):
    cid = lax.axis_index("c")
    sid = lax.axis_index("s")
    wid = sid * 2 + cid

    pltpu.sync_copy(w_hbm, w_v)
    half_wemb = w_v[0, :] * 0.5
    w_dense = w_v[1, :]
    bias_vec = w_v[2, :]
    lane = lax.iota(jnp.int32, 16)
    dcol = jnp.minimum(lane, NUM_DENSE - 1)
    bufs = (buf0, buf1)

    for c in range(NCH):
        sample0 = (wid * NCH + c) * CH

        pltpu.sync_copy(xs_hbm.at[pl.ds(sample0, CH), :], xs_v)
        pltpu.sync_copy(xd_hbm.at[pl.ds(sample0, CH), :], xd_v)

        def tr_body(g, _):
            rows = g * 16 + lane
            for f in range(N_FIELDS):
                ids = plsc.load_gather(xs_v, [rows, jnp.full((16,), f, jnp.int32)])
                idx_v[f, pl.ds(g * 16, 16)] = (
                    lax.shift_right_logical(ids, 3) + f * FIELD_STRIDE)
                off_v[f, pl.ds(g * 16, 16)] = (ids & 7) * EMB
            return 0

        lax.fori_loop(0, CH // 16, tr_body, 0)

        cp = pltpu.async_copy(tab_hbm.at[idx_v.at[0]], bufs[0], sem)
        for f in range(N_FIELDS):
            cp.wait()
            if f + 1 < N_FIELDS:
                cp = pltpu.async_copy(
                    tab_hbm.at[idx_v.at[f + 1]], bufs[(f + 1) % 2], sem)
            buf = bufs[f % 2]

            if f == 0:
                def acc_body0(j, _):
                    jv = jnp.full((16,), j, jnp.int32)
                    off = plsc.load_gather(off_v, [jnp.zeros((16,), jnp.int32), jv])
                    e = plsc.load_gather(buf0, [jv, off + lane])
                    s_v[j, :] = e
                    ss_v[j, :] = e * e
                    return 0
                lax.fori_loop(0, CH, acc_body0, 0)
            else:
                def acc_body(j, _, f=f, buf=buf):
                    jv = jnp.full((16,), j, jnp.int32)
                    off = plsc.load_gather(off_v, [jnp.full((16,), f, jnp.int32), jv])
                    e = plsc.load_gather(buf, [jv, off + lane])
                    s_v[j, :] = s_v[j, :] + e
                    ss_v[j, :] = ss_v[j, :] + e * e
                    return 0
                lax.fori_loop(0, CH, acc_body, 0)

        def group_body(g, _):
            def lane_body(l, _):
                j = g * 16 + l
                s = s_v[j, :]
                ss = ss_v[j, :]
                d = plsc.load_gather(xd_v, [jnp.full((16,), j, jnp.int32), dcol])
                t_v[l, :] = (s * s - ss) * half_wemb + d * w_dense
                return 0

            lax.fori_loop(0, 16, lane_body, 0)
            red = plsc.load_gather(t_v, [lane, jnp.zeros((16,), jnp.int32)])
            for d in range(1, EMB):
                red = red + plsc.load_gather(
                    t_v, [lane, jnp.full((16,), d, jnp.int32)])
            logits = red + bias_vec
            out_v[pl.ds(g * 16, 16)] = 1.0 / (1.0 + jnp.exp(-logits))
            return 0

        lax.fori_loop(0, CH // 16, group_body, 0)
        pltpu.sync_copy(out_v, out_hbm.at[pl.ds(sample0, CH)])


@jax.jit
def kernel(X_sparse, X_dense, tables, dnn_w, dnn_b):
    tabT = tables.transpose(0, 2, 1)
    w_emb = dnn_w[:EMB, 0]
    w_den = jnp.pad(dnn_w[EMB:, 0], (0, EMB - NUM_DENSE))
    b16 = jnp.broadcast_to(dnn_b, (EMB,))
    wcat = jnp.stack([w_emb, w_den, b16])

    mesh = plsc.VectorSubcoreMesh(core_axis_name="c", subcore_axis_name="s")

    tr_call = pl.kernel(
        _transpose_body,
        out_type=jax.ShapeDtypeStruct(
            (N_FIELDS * FIELD_STRIDE, LINE_W), jnp.float32),
        mesh=mesh,
        compiler_params=pltpu.CompilerParams(needs_layout_passes=False),
        scratch_types=[
            pltpu.VMEM((EMB, VCH), jnp.float32),
            pltpu.VMEM((VCH // ROWS_PER_LINE, LINE_W), jnp.float32),
            pltpu.VMEM((EMB, 2 * LINE_W), jnp.float32),
            pltpu.VMEM((24, LINE_W), jnp.float32),
            pltpu.SemaphoreType.DMA,
        ],
    )
    tab_lines = tr_call(tabT)

    call = pl.kernel(
        _sc_body,
        out_type=jax.ShapeDtypeStruct((BATCH,), jnp.float32),
        mesh=mesh,
        compiler_params=pltpu.CompilerParams(needs_layout_passes=False),
        scratch_types=[
            pltpu.VMEM((CH, N_FIELDS), jnp.int32),
            pltpu.VMEM((CH, NUM_DENSE), jnp.float32),
            pltpu.VMEM((N_FIELDS, CH), jnp.int32),
            pltpu.VMEM((N_FIELDS, CH), jnp.int32),
            pltpu.VMEM((CH, LINE_W), jnp.float32),
            pltpu.VMEM((CH, LINE_W), jnp.float32),
            pltpu.VMEM((CH, EMB), jnp.float32),
            pltpu.VMEM((CH, EMB), jnp.float32),
            pltpu.VMEM((3, EMB), jnp.float32),
            pltpu.VMEM((16, EMB), jnp.float32),
            pltpu.VMEM((CH,), jnp.float32),
            pltpu.SemaphoreType.DMA,
        ],
    )
    out = call(tab_lines, X_sparse.astype(jnp.int32), X_dense, wcat)
    return out.reshape(BATCH, 1)


# speedup vs baseline: 1.1709x; 1.1709x over previous
import jax
import jax.numpy as jnp
from jax import lax
from jax.experimental import pallas as pl
from jax.experimental.pallas import tpu as pltpu
from jax.experimental.pallas import tpu_sc as plsc

N_FIELDS = 26
VOCAB = 100000
EMB = 16
NUM_DENSE = 13
BATCH = 16384

ROWS_PER_LINE = 8
LINES_PER_FIELD = VOCAB // ROWS_PER_LINE
FIELD_STRIDE = 12512
LINE_W = ROWS_PER_LINE * EMB

NW = 32
SPW = BATCH // NW
CH = 128
NCH = SPW // CH

VCH = 1024
LCH = VCH // ROWS_PER_LINE
NFULL = VOCAB // VCH
VTAIL_READ = 768
LTAIL = VTAIL_READ // ROWS_PER_LINE
TOTAL_FULL = N_FIELDS * NFULL
PIPE_BLOCKS = TOTAL_FULL // NW


def _transpose_body(tabT_hbm, lines_hbm, in0, in1, out0, out1, in_t, out_t,
                    semi0, semi1, semo0, semo1, semt):
    cid = lax.axis_index("c")
    sid = lax.axis_index("s")
    wid = sid * 2 + cid
    lane = lax.iota(jnp.int32, 16)
    ins = (in0, in1)
    outs = (out0, out1)
    semis = (semi0, semi1)
    semos = (semo0, semo1)

    def fb_parts(i):
        fb = wid + i * NW
        return fb // NFULL, fb % NFULL

    def in_slice(i):
        f, b = fb_parts(i)
        return tabT_hbm.at[f, :, pl.ds(b * VCH, VCH)]

    def out_slice(i):
        f, b = fb_parts(i)
        return lines_hbm.at[pl.ds(f * FIELD_STRIDE + b * LCH, LCH), :]

    def transpose_block(src, dst, nlines):
        def vloop(v8, _):
            for k in range(ROWS_PER_LINE):
                vals = plsc.load_gather(
                    src, [lane, jnp.full((16,), v8 * 8 + k, jnp.int32)])
                dst[v8, pl.ds(k * EMB, EMB)] = vals
            return 0
        lax.fori_loop(0, nlines, vloop, 0)

    pltpu.async_copy(in_slice(0), ins[0], semis[0])
    pltpu.async_copy(in_slice(1), ins[1], semis[1])

    def pair_body(i2, _):
        for p in range(2):
            i = i2 * 2 + p
            pltpu.make_async_copy(in_slice(i), ins[p], semis[p]).wait()

            @pl.when(i2 > 0)
            def _():
                pltpu.make_async_copy(outs[p], out_slice(i - 2),
                                      semos[p]).wait()

            transpose_block(ins[p], outs[p], LCH)
            pltpu.async_copy(outs[p], out_slice(i), semos[p])

            @pl.when(i + 2 < PIPE_BLOCKS)
            def _():
                pltpu.async_copy(in_slice(i + 2), ins[p], semis[p])
        return 0

    lax.fori_loop(0, PIPE_BLOCKS // 2, pair_body, 0)
    pltpu.make_async_copy(outs[0], out_slice(PIPE_BLOCKS - 2), semos[0]).wait()
    pltpu.make_async_copy(outs[1], out_slice(PIPE_BLOCKS - 1), semos[1]).wait()

    @pl.when(wid < TOTAL_FULL - PIPE_BLOCKS * NW)
    def _leftover():
        fb = PIPE_BLOCKS * NW + wid
        f = fb // NFULL
        b = fb % NFULL
        pltpu.sync_copy(tabT_hbm.at[f, :, pl.ds(b * VCH, VCH)], ins[0])
        transpose_block(ins[0], outs[0], LCH)
        pltpu.sync_copy(
            outs[0], lines_hbm.at[pl.ds(f * FIELD_STRIDE + b * LCH, LCH), :])

    @pl.when(wid < N_FIELDS)
    def _tail():
        v0 = NFULL * VCH + wid * 0
        pltpu.sync_copy(tabT_hbm.at[wid, :, pl.ds(v0, VTAIL_READ)], in_t)
        transpose_block(in_t, out_t, LTAIL)
        pltpu.sync_copy(
            out_t,
            lines_hbm.at[pl.ds(wid * FIELD_STRIDE + NFULL * LCH, LTAIL), :])


def _sc_body(tab_hbm, xs_hbm, xd_hbm, w_hbm, out_hbm,
             xs_v, xd_v, idx_v, off_v, buf0, buf1, s_v, ss_v, w_v, t_v, out_v,
             sem):
    cid = lax.axis_index("c")
    sid = lax.axis_index("s")
    wid = sid * 2 + cid

    pltpu.sync_copy(w_hbm, w_v)
    half_wemb = w_v[0, :] * 0.5
    w_dense = w_v[1, :]
    bias_vec = w_v[2, :]
    lane = lax.iota(jnp.int32, 16)
    dcol = jnp.minimum(lane, NUM_DENSE - 1)
    bufs = (buf0, buf1)

    for c in range(NCH):
        sample0 = (wid * NCH + c) * CH

        pltpu.sync_copy(xs_hbm.at[pl.ds(sample0, CH), :], xs_v)
        pltpu.sync_copy(xd_hbm.at[pl.ds(sample0, CH), :], xd_v)

        def tr_body(g, _):
            rows = g * 16 + lane
            for f in range(N_FIELDS):
                ids = plsc.load_gather(xs_v, [rows, jnp.full((16,), f, jnp.int32)])
                idx_v[f, pl.ds(g * 16, 16)] = (
                    lax.shift_right_logical(ids, 3) + f * FIELD_STRIDE)
                off_v[f, pl.ds(g * 16, 16)] = (ids & 7) * EMB
            return 0

        lax.fori_loop(0, CH // 16, tr_body, 0)

        cp = pltpu.async_copy(tab_hbm.at[idx_v.at[0]], bufs[0], sem)
        for f in range(N_FIELDS):
            cp.wait()
            if f + 1 < N_FIELDS:
                cp = pltpu.async_copy(
                    tab_hbm.at[idx_v.at[f + 1]], bufs[(f + 1) % 2], sem)
            buf = bufs[f % 2]

            if f == 0:
                def acc_body0(j, _):
                    jv = jnp.full((16,), j, jnp.int32)
                    off = plsc.load_gather(off_v, [jnp.zeros((16,), jnp.int32), jv])
                    e = plsc.load_gather(buf0, [jv, off + lane])
                    s_v[j, :] = e
                    ss_v[j, :] = e * e
                    return 0
                lax.fori_loop(0, CH, acc_body0, 0)
            else:
                def acc_body(j, _, f=f, buf=buf):
                    jv = jnp.full((16,), j, jnp.int32)
                    off = plsc.load_gather(off_v, [jnp.full((16,), f, jnp.int32), jv])
                    e = plsc.load_gather(buf, [jv, off + lane])
                    s_v[j, :] = s_v[j, :] + e
                    ss_v[j, :] = ss_v[j, :] + e * e
                    return 0
                lax.fori_loop(0, CH, acc_body, 0)

        def group_body(g, _):
            def lane_body(l, _):
                j = g * 16 + l
                s = s_v[j, :]
                ss = ss_v[j, :]
                d = plsc.load_gather(xd_v, [jnp.full((16,), j, jnp.int32), dcol])
                t_v[l, :] = (s * s - ss) * half_wemb + d * w_dense
                return 0

            lax.fori_loop(0, 16, lane_body, 0)
            red = plsc.load_gather(t_v, [lane, jnp.zeros((16,), jnp.int32)])
            for d in range(1, EMB):
                red = red + plsc.load_gather(
                    t_v, [lane, jnp.full((16,), d, jnp.int32)])
            logits = red + bias_vec
            out_v[pl.ds(g * 16, 16)] = 1.0 / (1.0 + jnp.exp(-logits))
            return 0

        lax.fori_loop(0, CH // 16, group_body, 0)
        pltpu.sync_copy(out_v, out_hbm.at[pl.ds(sample0, CH)])


@jax.jit
def kernel(X_sparse, X_dense, tables, dnn_w, dnn_b):
    tabT = tables.transpose(0, 2, 1)
    w_emb = dnn_w[:EMB, 0]
    w_den = jnp.pad(dnn_w[EMB:, 0], (0, EMB - NUM_DENSE))
    b16 = jnp.broadcast_to(dnn_b, (EMB,))
    wcat = jnp.stack([w_emb, w_den, b16])

    mesh = plsc.VectorSubcoreMesh(core_axis_name="c", subcore_axis_name="s")

    tr_call = pl.kernel(
        _transpose_body,
        out_type=jax.ShapeDtypeStruct(
            (N_FIELDS * FIELD_STRIDE, LINE_W), jnp.float32),
        mesh=mesh,
        compiler_params=pltpu.CompilerParams(needs_layout_passes=False),
        scratch_types=[
            pltpu.VMEM((EMB, VCH), jnp.float32),
            pltpu.VMEM((EMB, VCH), jnp.float32),
            pltpu.VMEM((LCH, LINE_W), jnp.float32),
            pltpu.VMEM((LCH, LINE_W), jnp.float32),
            pltpu.VMEM((EMB, VTAIL_READ), jnp.float32),
            pltpu.VMEM((LTAIL, LINE_W), jnp.float32),
            pltpu.SemaphoreType.DMA,
            pltpu.SemaphoreType.DMA,
            pltpu.SemaphoreType.DMA,
            pltpu.SemaphoreType.DMA,
            pltpu.SemaphoreType.DMA,
        ],
    )
    tab_lines = tr_call(tabT)

    call = pl.kernel(
        _sc_body,
        out_type=jax.ShapeDtypeStruct((BATCH,), jnp.float32),
        mesh=mesh,
        compiler_params=pltpu.CompilerParams(needs_layout_passes=False),
        scratch_types=[
            pltpu.VMEM((CH, N_FIELDS), jnp.int32),
            pltpu.VMEM((CH, NUM_DENSE), jnp.float32),
            pltpu.VMEM((N_FIELDS, CH), jnp.int32),
            pltpu.VMEM((N_FIELDS, CH), jnp.int32),
            pltpu.VMEM((CH, LINE_W), jnp.float32),
            pltpu.VMEM((CH, LINE_W), jnp.float32),
            pltpu.VMEM((CH, EMB), jnp.float32),
            pltpu.VMEM((CH, EMB), jnp.float32),
            pltpu.VMEM((3, EMB), jnp.float32),
            pltpu.VMEM((16, EMB), jnp.float32),
            pltpu.VMEM((CH,), jnp.float32),
            pltpu.SemaphoreType.DMA,
        ],
    )
    out = call(tab_lines, X_sparse.astype(jnp.int32), X_dense, wcat)
    return out.reshape(BATCH, 1)


# speedup vs baseline: 1.3381x; 1.1429x over previous
import jax
import jax.numpy as jnp
from jax import lax
from jax.experimental import pallas as pl
from jax.experimental.pallas import tpu as pltpu
from jax.experimental.pallas import tpu_sc as plsc

N_FIELDS = 26
VOCAB = 100000
EMB = 16
NUM_DENSE = 13
BATCH = 16384

ROWS_PER_LINE = 8
FIELD_STRIDE = 12512
LINE_W = ROWS_PER_LINE * EMB

NW = 32
SPW = BATCH // NW
CH = 128
NCH = SPW // CH

VCH = 1024
VPAD = VCH + 1
LCH = VCH // ROWS_PER_LINE
NFULL = VOCAB // VCH
VTAIL_READ = 768
LTAIL = VTAIL_READ // ROWS_PER_LINE
TOTAL_FULL = N_FIELDS * NFULL
PIPE_BLOCKS = TOTAL_FULL // NW


def _transpose_body(tabT_hbm, lines_hbm, in0, in1, out0, out1, in_t, out_t,
                    semi0, semi1, semo0, semo1):
    cid = lax.axis_index("c")
    sid = lax.axis_index("s")
    wid = sid * 2 + cid
    lane = lax.iota(jnp.int32, 16)
    ins = (in0, in1)
    outs = (out0, out1)
    semis = (semi0, semi1)
    semos = (semo0, semo1)

    def fb_parts(i):
        fb = wid + i * NW
        return fb // NFULL, fb % NFULL

    def in_slice(i):
        f, b = fb_parts(i)
        return tabT_hbm.at[f, :, pl.ds(b * VCH, VCH)]

    def out_slice(i):
        f, b = fb_parts(i)
        return lines_hbm.at[pl.ds(f * FIELD_STRIDE + b * LCH, LCH), :]

    def transpose_block(src, dst, nlines):
        def vloop(v8, _):
            for k in range(ROWS_PER_LINE):
                vals = plsc.load_gather(
                    src, [lane, jnp.full((16,), v8 * 8 + k, jnp.int32)])
                dst[v8, pl.ds(k * EMB, EMB)] = vals
            return 0
        lax.fori_loop(0, nlines, vloop, 0)

    def in_dst(p):
        return ins[p].at[:, pl.ds(0, VCH)]

    pltpu.async_copy(in_slice(0), in_dst(0), semis[0])
    pltpu.async_copy(in_slice(1), in_dst(1), semis[1])

    def pair_body(i2, _):
        for p in range(2):
            i = i2 * 2 + p
            pltpu.make_async_copy(in_slice(i), in_dst(p), semis[p]).wait()

            @pl.when(i2 > 0)
            def _():
                pltpu.make_async_copy(outs[p], out_slice(i - 2),
                                      semos[p]).wait()

            transpose_block(ins[p], outs[p], LCH)
            pltpu.async_copy(outs[p], out_slice(i), semos[p])

            @pl.when(i + 2 < PIPE_BLOCKS)
            def _():
                pltpu.async_copy(in_slice(i + 2), in_dst(p), semis[p])
        return 0

    lax.fori_loop(0, PIPE_BLOCKS // 2, pair_body, 0)
    pltpu.make_async_copy(outs[0], out_slice(PIPE_BLOCKS - 2), semos[0]).wait()
    pltpu.make_async_copy(outs[1], out_slice(PIPE_BLOCKS - 1), semos[1]).wait()

    @pl.when(wid < TOTAL_FULL - PIPE_BLOCKS * NW)
    def _leftover():
        fb = PIPE_BLOCKS * NW + wid
        f = fb // NFULL
        b = fb % NFULL
        pltpu.sync_copy(tabT_hbm.at[f, :, pl.ds(b * VCH, VCH)], in_dst(0))
        transpose_block(ins[0], outs[0], LCH)
        pltpu.sync_copy(
            outs[0], lines_hbm.at[pl.ds(f * FIELD_STRIDE + b * LCH, LCH), :])

    @pl.when(wid < N_FIELDS)
    def _tail():
        v0 = NFULL * VCH + wid * 0
        pltpu.sync_copy(tabT_hbm.at[wid, :, pl.ds(v0, VTAIL_READ)],
                        in_t.at[:, pl.ds(0, VTAIL_READ)])
        transpose_block(in_t, out_t, LTAIL)
        pltpu.sync_copy(
            out_t,
            lines_hbm.at[pl.ds(wid * FIELD_STRIDE + NFULL * LCH, LTAIL), :])


def _sc_body(tab_hbm, xs_hbm, xd_hbm, w_hbm, out_hbm,
             xs_v, xd_v, idx_v, off_v, buf0, buf1, s_m, ss_m, w_v, out_v,
             sem0, sem1):
    cid = lax.axis_index("c")
    sid = lax.axis_index("s")
    wid = sid * 2 + cid

    pltpu.sync_copy(w_hbm, w_v)
    lane = lax.iota(jnp.int32, 16)
    rots = [(lane + k) & 15 for k in range(EMB)]
    half_w = [plsc.load_gather(w_v, [jnp.zeros((16,), jnp.int32), rots[k]])
              for k in range(EMB)]
    wd_b = [plsc.load_gather(w_v, [jnp.ones((16,), jnp.int32),
                                   jnp.full((16,), dd, jnp.int32)])
            for dd in range(NUM_DENSE)]
    bias_vec = w_v[2, :]
    bufs = (buf0, buf1)
    sems = (sem0, sem1)

    def chunk_body(c, _):
        sample0 = (wid * NCH + c) * CH

        pltpu.sync_copy(xs_hbm.at[pl.ds(sample0, CH), :], xs_v)
        pltpu.sync_copy(xd_hbm.at[pl.ds(sample0, CH), :], xd_v)

        def tr_body(g, _):
            rows = g * 16 + lane
            for f in range(N_FIELDS):
                ids = plsc.load_gather(xs_v, [rows, jnp.full((16,), f, jnp.int32)])
                idx_v[f, pl.ds(g * 16, 16)] = (
                    lax.shift_right_logical(ids, 3) + f * FIELD_STRIDE)
                off_v[f, pl.ds(g * 16, 16)] = (ids & 7) * EMB
            return 0

        lax.fori_loop(0, CH // 16, tr_body, 0)

        def zinit(g, _):
            z = jnp.zeros((16,), jnp.float32)
            for k in range(EMB):
                s_m[k, pl.ds(g * 16, 16)] = z
                ss_m[k, pl.ds(g * 16, 16)] = z
            return 0

        lax.fori_loop(0, CH // 16, zinit, 0)

        pltpu.async_copy(tab_hbm.at[idx_v.at[0]], buf0, sem0)
        pltpu.async_copy(tab_hbm.at[idx_v.at[1]], buf1, sem1)

        def field_pair(f2, _):
            for p in range(2):
                f = f2 * 2 + p
                buf, semp = bufs[p], sems[p]
                pltpu.make_async_copy(
                    tab_hbm.at[idx_v.at[f]], buf, semp).wait()

                def acc_body(g, _, buf=buf, f=f):
                    rows = g * 16 + lane
                    offs = off_v[f, pl.ds(g * 16, 16)]
                    for k in range(EMB):
                        e = plsc.load_gather(buf, [rows, offs + rots[k]])
                        plsc.addupdate(s_m.at[k, pl.ds(g * 16, 16)], e)
                        plsc.addupdate(ss_m.at[k, pl.ds(g * 16, 16)], e * e)
                    return 0

                lax.fori_loop(0, CH // 16, acc_body, 0)

                @pl.when(f + 2 < N_FIELDS)
                def _(buf=buf, semp=semp, f=f):
                    pltpu.async_copy(tab_hbm.at[idx_v.at[f + 2]], buf, semp)
            return 0

        lax.fori_loop(0, N_FIELDS // 2, field_pair, 0)

        def fin_body(g, _):
            rows = g * 16 + lane
            acc = jnp.zeros((16,), jnp.float32)
            for k in range(EMB):
                s = s_m[k, pl.ds(g * 16, 16)]
                ss = ss_m[k, pl.ds(g * 16, 16)]
                acc = acc + (s * s - ss) * half_w[k]
            acc = acc * 0.5
            for dd in range(NUM_DENSE):
                d = plsc.load_gather(xd_v, [rows, jnp.full((16,), dd, jnp.int32)])
                acc = acc + d * wd_b[dd]
            logits = acc + bias_vec
            out_v[pl.ds(g * 16, 16)] = 1.0 / (1.0 + jnp.exp(-logits))
            return 0

        lax.fori_loop(0, CH // 16, fin_body, 0)
        pltpu.sync_copy(out_v, out_hbm.at[pl.ds(sample0, CH)])
        return 0

    lax.fori_loop(0, NCH, chunk_body, 0)


@jax.jit
def kernel(X_sparse, X_dense, tables, dnn_w, dnn_b):
    tabT = tables.transpose(0, 2, 1)
    w_emb = dnn_w[:EMB, 0]
    w_den = jnp.pad(dnn_w[EMB:, 0], (0, EMB - NUM_DENSE))
    b16 = jnp.broadcast_to(dnn_b, (EMB,))
    wcat = jnp.stack([w_emb, w_den, b16])

    mesh = plsc.VectorSubcoreMesh(core_axis_name="c", subcore_axis_name="s")

    tr_call = pl.kernel(
        _transpose_body,
        out_type=jax.ShapeDtypeStruct(
            (N_FIELDS * FIELD_STRIDE, LINE_W), jnp.float32),
        mesh=mesh,
        compiler_params=pltpu.CompilerParams(needs_layout_passes=False),
        scratch_types=[
            pltpu.VMEM((EMB, VPAD), jnp.float32),
            pltpu.VMEM((EMB, VPAD), jnp.float32),
            pltpu.VMEM((LCH, LINE_W), jnp.float32),
            pltpu.VMEM((LCH, LINE_W), jnp.float32),
            pltpu.VMEM((EMB, VTAIL_READ + 1), jnp.float32),
            pltpu.VMEM((LTAIL, LINE_W), jnp.float32),
            pltpu.SemaphoreType.DMA,
            pltpu.SemaphoreType.DMA,
            pltpu.SemaphoreType.DMA,
            pltpu.SemaphoreType.DMA,
        ],
    )
    tab_lines = tr_call(tabT)

    call = pl.kernel(
        _sc_body,
        out_type=jax.ShapeDtypeStruct((BATCH,), jnp.float32),
        mesh=mesh,
        compiler_params=pltpu.CompilerParams(needs_layout_passes=False),
        scratch_types=[
            pltpu.VMEM((CH, N_FIELDS), jnp.int32),
            pltpu.VMEM((CH, NUM_DENSE), jnp.float32),
            pltpu.VMEM((N_FIELDS, CH), jnp.int32),
            pltpu.VMEM((N_FIELDS, CH), jnp.int32),
            pltpu.VMEM((CH, LINE_W), jnp.float32),
            pltpu.VMEM((CH, LINE_W), jnp.float32),
            pltpu.VMEM((EMB, CH), jnp.float32),
            pltpu.VMEM((EMB, CH), jnp.float32),
            pltpu.VMEM((3, EMB), jnp.float32),
            pltpu.VMEM((CH,), jnp.float32),
            pltpu.SemaphoreType.DMA,
            pltpu.SemaphoreType.DMA,
        ],
    )
    out = call(tab_lines, X_sparse.astype(jnp.int32), X_dense, wcat)
    return out.reshape(BATCH, 1)


# speedup vs baseline: 2.0347x; 1.5206x over previous
import jax
import jax.numpy as jnp
from jax import lax
from jax.experimental import pallas as pl
from jax.experimental.pallas import tpu as pltpu
from jax.experimental.pallas import tpu_sc as plsc

N_FIELDS = 26
VOCAB = 100000
EMB = 16
NUM_DENSE = 13
BATCH = 16384

ROWS_PER_LINE = 8
FIELD_STRIDE = 12512
LINE_W = ROWS_PER_LINE * EMB

NW = 32
SPW = BATCH // NW
CH = 128
NCH = SPW // CH

VCH = 1024
VPAD = VCH + 1
LCH = VCH // ROWS_PER_LINE
NFULL = VOCAB // VCH
VTAIL_READ = 768
LTAIL = VTAIL_READ // ROWS_PER_LINE
TOTAL_FULL = N_FIELDS * NFULL
PIPE_BLOCKS = TOTAL_FULL // NW


def _transpose_body(tabT_hbm, lines_hbm, in0, in1, out0, out1, in_t, out_t,
                    semi0, semi1, semo0, semo1):
    cid = lax.axis_index("c")
    sid = lax.axis_index("s")
    wid = sid * 2 + cid
    lane = lax.iota(jnp.int32, 16)
    ins = (in0, in1)
    outs = (out0, out1)
    semis = (semi0, semi1)
    semos = (semo0, semo1)

    def fb_parts(i):
        fb = wid + i * NW
        return fb // NFULL, fb % NFULL

    def in_slice(i):
        f, b = fb_parts(i)
        return tabT_hbm.at[f, :, pl.ds(b * VCH, VCH)]

    def out_slice(i):
        f, b = fb_parts(i)
        return lines_hbm.at[pl.ds(f * FIELD_STRIDE + b * LCH, LCH), :]

    def transpose_block(src, dst, nlines):
        @plsc.parallel_loop(0, nlines, unroll=2)
        def vloop(v8):
            vals = [plsc.load_gather(
                src, [lane, jnp.full((16,), v8 * 8 + k, jnp.int32)])
                for k in range(ROWS_PER_LINE)]
            for k in range(ROWS_PER_LINE):
                dst[v8, pl.ds(k * EMB, EMB)] = vals[k]

    def in_dst(p):
        return ins[p].at[:, pl.ds(0, VCH)]

    pltpu.async_copy(in_slice(0), in_dst(0), semis[0])
    pltpu.async_copy(in_slice(1), in_dst(1), semis[1])

    def pair_body(i2, _):
        for p in range(2):
            i = i2 * 2 + p
            pltpu.make_async_copy(in_slice(i), in_dst(p), semis[p]).wait()

            @pl.when(i2 > 0)
            def _():
                pltpu.make_async_copy(outs[p], out_slice(i - 2),
                                      semos[p]).wait()

            transpose_block(ins[p], outs[p], LCH)
            pltpu.async_copy(outs[p], out_slice(i), semos[p])

            @pl.when(i + 2 < PIPE_BLOCKS)
            def _():
                pltpu.async_copy(in_slice(i + 2), in_dst(p), semis[p])
        return 0

    lax.fori_loop(0, PIPE_BLOCKS // 2, pair_body, 0)
    pltpu.make_async_copy(outs[0], out_slice(PIPE_BLOCKS - 2), semos[0]).wait()
    pltpu.make_async_copy(outs[1], out_slice(PIPE_BLOCKS - 1), semos[1]).wait()

    @pl.when(wid < TOTAL_FULL - PIPE_BLOCKS * NW)
    def _leftover():
        fb = PIPE_BLOCKS * NW + wid
        f = fb // NFULL
        b = fb % NFULL
        pltpu.sync_copy(tabT_hbm.at[f, :, pl.ds(b * VCH, VCH)], in_dst(0))
        transpose_block(ins[0], outs[0], LCH)
        pltpu.sync_copy(
            outs[0], lines_hbm.at[pl.ds(f * FIELD_STRIDE + b * LCH, LCH), :])

    @pl.when(wid < N_FIELDS)
    def _tail():
        v0 = NFULL * VCH + wid * 0
        pltpu.sync_copy(tabT_hbm.at[wid, :, pl.ds(v0, VTAIL_READ)],
                        in_t.at[:, pl.ds(0, VTAIL_READ)])
        transpose_block(in_t, out_t, LTAIL)
        pltpu.sync_copy(
            out_t,
            lines_hbm.at[pl.ds(wid * FIELD_STRIDE + NFULL * LCH, LTAIL), :])


def _sc_body(tab_hbm, xs_hbm, xd_hbm, w_hbm, out_hbm,
             xs_v, xd_v, idx_v, off_v, buf0, buf1, s_m, ss_m, w_v, out_v,
             sem0, sem1):
    cid = lax.axis_index("c")
    sid = lax.axis_index("s")
    wid = sid * 2 + cid

    pltpu.sync_copy(w_hbm, w_v)
    lane = lax.iota(jnp.int32, 16)
    rots = [(lane + k) & 15 for k in range(EMB)]
    half_w = [plsc.load_gather(w_v, [jnp.zeros((16,), jnp.int32), rots[k]])
              for k in range(EMB)]
    wd_b = [plsc.load_gather(w_v, [jnp.ones((16,), jnp.int32),
                                   jnp.full((16,), dd, jnp.int32)])
            for dd in range(NUM_DENSE)]
    bias_vec = w_v[2, :]
    bufs = (buf0, buf1)
    sems = (sem0, sem1)

    def chunk_body(c, _):
        sample0 = (wid * NCH + c) * CH

        pltpu.sync_copy(xs_hbm.at[pl.ds(sample0, CH), :], xs_v)
        pltpu.sync_copy(xd_hbm.at[pl.ds(sample0, CH), :], xd_v)

        def tr_body(g, _):
            rows = g * 16 + lane
            for f in range(N_FIELDS):
                ids = plsc.load_gather(xs_v, [rows, jnp.full((16,), f, jnp.int32)])
                idx_v[f, pl.ds(g * 16, 16)] = (
                    lax.shift_right_logical(ids, 3) + f * FIELD_STRIDE)
                off_v[f, pl.ds(g * 16, 16)] = (ids & 7) * EMB
            return 0

        lax.fori_loop(0, CH // 16, tr_body, 0)

        def zinit(g, _):
            z = jnp.zeros((16,), jnp.float32)
            for k in range(EMB):
                s_m[k, pl.ds(g * 16, 16)] = z
                ss_m[k, pl.ds(g * 16, 16)] = z
            return 0

        lax.fori_loop(0, CH // 16, zinit, 0)

        pltpu.async_copy(tab_hbm.at[idx_v.at[0]], buf0, sem0)
        pltpu.async_copy(tab_hbm.at[idx_v.at[1]], buf1, sem1)

        def field_pair(f2, _):
            for p in range(2):
                f = f2 * 2 + p
                buf, semp = bufs[p], sems[p]
                pltpu.make_async_copy(
                    tab_hbm.at[idx_v.at[f]], buf, semp).wait()

                def acc_body(g, _, buf=buf, f=f):
                    rows = g * 16 + lane
                    offs = off_v[f, pl.ds(g * 16, 16)]
                    for k in range(EMB):
                        e = plsc.load_gather(buf, [rows, offs + rots[k]])
                        plsc.addupdate(s_m.at[k, pl.ds(g * 16, 16)], e)
                        plsc.addupdate(ss_m.at[k, pl.ds(g * 16, 16)], e * e)
                    return 0

                lax.fori_loop(0, CH // 16, acc_body, 0)

                @pl.when(f + 2 < N_FIELDS)
                def _(buf=buf, semp=semp, f=f):
                    pltpu.async_copy(tab_hbm.at[idx_v.at[f + 2]], buf, semp)
            return 0

        lax.fori_loop(0, N_FIELDS // 2, field_pair, 0)

        def fin_body(g, _):
            rows = g * 16 + lane
            acc = jnp.zeros((16,), jnp.float32)
            for k in range(EMB):
                s = s_m[k, pl.ds(g * 16, 16)]
                ss = ss_m[k, pl.ds(g * 16, 16)]
                acc = acc + (s * s - ss) * half_w[k]
            acc = acc * 0.5
            for dd in range(NUM_DENSE):
                d = plsc.load_gather(xd_v, [rows, jnp.full((16,), dd, jnp.int32)])
                acc = acc + d * wd_b[dd]
            logits = acc + bias_vec
            out_v[pl.ds(g * 16, 16)] = 1.0 / (1.0 + jnp.exp(-logits))
            return 0

        lax.fori_loop(0, CH // 16, fin_body, 0)
        pltpu.sync_copy(out_v, out_hbm.at[pl.ds(sample0, CH)])
        return 0

    lax.fori_loop(0, NCH, chunk_body, 0)


@jax.jit
def kernel(X_sparse, X_dense, tables, dnn_w, dnn_b):
    tabT = tables.transpose(0, 2, 1)
    w_emb = dnn_w[:EMB, 0]
    w_den = jnp.pad(dnn_w[EMB:, 0], (0, EMB - NUM_DENSE))
    b16 = jnp.broadcast_to(dnn_b, (EMB,))
    wcat = jnp.stack([w_emb, w_den, b16])

    mesh = plsc.VectorSubcoreMesh(core_axis_name="c", subcore_axis_name="s")

    tr_call = pl.kernel(
        _transpose_body,
        out_type=jax.ShapeDtypeStruct(
            (N_FIELDS * FIELD_STRIDE, LINE_W), jnp.float32),
        mesh=mesh,
        compiler_params=pltpu.CompilerParams(needs_layout_passes=False),
        scratch_types=[
            pltpu.VMEM((EMB, VPAD), jnp.float32),
            pltpu.VMEM((EMB, VPAD), jnp.float32),
            pltpu.VMEM((LCH, LINE_W), jnp.float32),
            pltpu.VMEM((LCH, LINE_W), jnp.float32),
            pltpu.VMEM((EMB, VTAIL_READ + 1), jnp.float32),
            pltpu.VMEM((LTAIL, LINE_W), jnp.float32),
            pltpu.SemaphoreType.DMA,
            pltpu.SemaphoreType.DMA,
            pltpu.SemaphoreType.DMA,
            pltpu.SemaphoreType.DMA,
        ],
    )
    tab_lines = tr_call(tabT)

    call = pl.kernel(
        _sc_body,
        out_type=jax.ShapeDtypeStruct((BATCH,), jnp.float32),
        mesh=mesh,
        compiler_params=pltpu.CompilerParams(needs_layout_passes=False),
        scratch_types=[
            pltpu.VMEM((CH, N_FIELDS), jnp.int32),
            pltpu.VMEM((CH, NUM_DENSE), jnp.float32),
            pltpu.VMEM((N_FIELDS, CH), jnp.int32),
            pltpu.VMEM((N_FIELDS, CH), jnp.int32),
            pltpu.VMEM((CH, LINE_W), jnp.float32),
            pltpu.VMEM((CH, LINE_W), jnp.float32),
            pltpu.VMEM((EMB, CH), jnp.float32),
            pltpu.VMEM((EMB, CH), jnp.float32),
            pltpu.VMEM((3, EMB), jnp.float32),
            pltpu.VMEM((CH,), jnp.float32),
            pltpu.SemaphoreType.DMA,
            pltpu.SemaphoreType.DMA,
        ],
    )
    out = call(tab_lines, X_sparse.astype(jnp.int32), X_dense, wcat)
    return out.reshape(BATCH, 1)
